# baseline (speedup 1.0000x reference)
import functools

import jax
import jax.numpy as jnp
from jax import lax
from jax.experimental import pallas as pl
from jax.experimental.pallas import tpu as pltpu
from jax.experimental.pallas import tpu_sc as plsc

N_NODES = 10000
N_EDGES = 160000
IN_FEATS = 256
N_HIDDEN = 512
N_CLASSES = 64

NC = 2
NS = 16
NB = 125
N_PAD = 10240
STRIPE = N_PAD // NS

ROW_BLK = 1024
N_BLKS = N_PAD // ROW_BLK

_SC_MESH = dict(core_axis_name="c", subcore_axis_name="s")


def _deg_kernel_body(src_t, dst_t, zeros_hbm, out_ref, idx_v, ones_v, hist_sp):
    c = lax.axis_index("c")
    s = lax.axis_index("s")

    @pl.when(c == 0)
    def _():
        pltpu.sync_copy(src_t.at[s], idx_v)

    @pl.when(c == 1)
    def _():
        pltpu.sync_copy(dst_t.at[s], idx_v)

    def fill_ones(i, _):
        ones_v[pl.ds(16 * i, 16)] = jnp.full((16,), 1.0, dtype=jnp.float32)
        return 0

    lax.fori_loop(0, 80 // 16, fill_ones, 0)

    pltpu.sync_copy(zeros_hbm, hist_sp.at[pl.ds(STRIPE * s, STRIPE)])
    plsc.subcore_barrier()

    def add_block(j, _):
        pltpu.sync_copy(ones_v, hist_sp.at[idx_v.at[j]], add=True)
        return 0

    lax.fori_loop(0, NB, add_block, 0)
    plsc.subcore_barrier()
    pltpu.sync_copy(hist_sp.at[pl.ds(STRIPE * s, STRIPE)],
                    out_ref.at[c].at[pl.ds(STRIPE * s, STRIPE)])


_deg_kernel = functools.partial(
    pl.kernel,
    out_type=jax.ShapeDtypeStruct((NC, N_PAD), jnp.float32),
    mesh=plsc.VectorSubcoreMesh(**_SC_MESH),
    scratch_types=[
        pltpu.VMEM((NB, 80), jnp.int32),
        pltpu.VMEM((80,), jnp.float32),
        pltpu.VMEM_SHARED((N_PAD,), jnp.float32),
    ],
)(_deg_kernel_body)


def _make_agg(P, split):
    W = 128
    R = 1 if split else P // NC
    BB = 40 if split else 80
    NBK = 125
    K = 4
    G = 2
    S = 1
    I = 3

    def body(h_ref, sd_ref, zeros_hbm, out_ref, idx_v, gbuf, acc,
             isem, gsem, ssem):
        c = lax.axis_index("c")
        s = lax.axis_index("s")
        sd_me = sd_ref.at[c].at[s] if split else sd_ref.at[s]

        def idx_start(j, buf):
            pltpu.make_async_copy(sd_me.at[j], idx_v.at[buf],
                                  isem.at[buf]).start()

        def idx_wait(j, buf):
            pltpu.make_async_copy(sd_me.at[j], idx_v.at[buf],
                                  isem.at[buf]).wait()

        for r in range(R):
            slab = 0 if split else NC * r + c
            pltpu.sync_copy(zeros_hbm, acc.at[pl.ds(STRIPE * s, STRIPE)])
            plsc.subcore_barrier()

            h_slab = h_ref.at[slab]

            def gather_start(j, buf):
                pltpu.make_async_copy(
                    h_slab.at[idx_v.at[buf].at[0]],
                    gbuf.at[buf], gsem.at[buf]).start()

            def gather_wait(j, buf):
                pltpu.make_async_copy(
                    h_slab.at[idx_v.at[buf].at[0]],
                    gbuf.at[buf], gsem.at[buf]).wait()

            def scatter_start(j, buf):
                pltpu.async_copy(gbuf.at[buf], acc.at[idx_v.at[buf].at[1]],
                                 ssem.at[buf], add=True)

            def scatter_wait(j, buf):
                pltpu.make_async_copy(gbuf.at[buf], acc.at[idx_v.at[buf].at[1]],
                                      ssem.at[buf]).wait()

            for t in range(I):
                idx_start(t, t)
            for t in range(G):
                idx_wait(t, t)
                gather_start(t, t)

            def step(j, _):
                buf = lax.rem(j, K)
                gather_wait(j, buf)
                scatter_start(j, buf)

                @pl.when(j >= S)
                def _():
                    scatter_wait(j - S, lax.rem(j + K - S, K))

                @pl.when(j + G < NBK)
                def _():
                    bg = lax.rem(j + G, K)
                    idx_wait(j + G, bg)
                    gather_start(j + G, bg)

                @pl.when(j + I < NBK)
                def _():
                    idx_start(j + I, lax.rem(j + I, K))

                return 0

            lax.fori_loop(0, NBK, step, 0)
            for t in range(S):
                scatter_wait(NBK - S + t, (NBK - S + t) % K)
            plsc.subcore_barrier()
            dst_slab = out_ref.at[c] if split else out_ref.at[slab]
            pltpu.sync_copy(acc.at[pl.ds(STRIPE * s, STRIPE)],
                            dst_slab.at[pl.ds(STRIPE * s, STRIPE)])
            if r + 1 < R:
                plsc.subcore_barrier()

    out_slabs = NC if split else P
    idx_shape = (NC, NS) if split else (NS,)
    return functools.partial(
        pl.kernel,
        out_type=jax.ShapeDtypeStruct((out_slabs, N_PAD, W), jnp.float32),
        mesh=plsc.VectorSubcoreMesh(**_SC_MESH),
        scratch_types=[
            pltpu.VMEM((K, 2, BB), jnp.int32),
            pltpu.VMEM((K, BB, W), jnp.float32),
            pltpu.VMEM_SHARED((N_PAD, W), jnp.float32),
            pltpu.SemaphoreType.DMA((K,)),
            pltpu.SemaphoreType.DMA((K,)),
            pltpu.SemaphoreType.DMA((K,)),
        ],
    )(body)


_agg_256 = _make_agg(2, split=False)
_agg_512 = _make_agg(4, split=False)
_agg_64 = _make_agg(1, split=True)


def _norms(deg_ref):
    dsrc = deg_ref[0, :]
    ddst = deg_ref[1, :]
    ns = jnp.where(dsrc > 0, lax.rsqrt(jnp.maximum(dsrc, 1.0)), 0.0)
    nd = jnp.where(ddst > 0, lax.rsqrt(jnp.maximum(ddst, 1.0)), 0.0)
    return ns, nd


def _tc_a_body(deg_ref, feat_ref, out_ref):
    ns, _ = _norms(deg_ref)
    x = feat_ref[...] * ns[:, None]
    out_ref[0] = x[:, 0:128]
    out_ref[1] = x[:, 128:256]


def _tc_b_body(deg_ref, agg_ref, w0_ref, b0_ref, out_ref):
    ns, nd = _norms(deg_ref)
    acc = jnp.dot(agg_ref[0], w0_ref[0:128, :], preferred_element_type=jnp.float32)
    acc += jnp.dot(agg_ref[1], w0_ref[128:256, :], preferred_element_type=jnp.float32)
    h1 = jnp.maximum(acc * nd[:, None] + b0_ref[...], 0.0) * ns[:, None]
    for q in range(4):
        out_ref[q] = h1[:, 128 * q:128 * (q + 1)]


def _tc_c_body(deg_ref, agg_ref, w1_ref, b1_ref, w2_ref, out_ref):
    ns, nd = _norms(deg_ref)
    acc = jnp.dot(agg_ref[0], w1_ref[0:128, :], preferred_element_type=jnp.float32)
    for p in range(1, 4):
        acc += jnp.dot(agg_ref[p], w1_ref[128 * p:128 * (p + 1), :],
                       preferred_element_type=jnp.float32)
    t = jnp.maximum(acc * nd[:, None] + b1_ref[...], 0.0) * ns[:, None]
    z = jnp.dot(t, w2_ref[...], preferred_element_type=jnp.float32)
    out_ref[0] = jnp.concatenate([z, jnp.zeros_like(z)], axis=1)


def _tc_d_body(deg_ref, agg_ref, b2_ref, out_ref):
    _, nd = _norms(deg_ref)
    y = agg_ref[0, :, 0:N_CLASSES] + agg_ref[1, :, 0:N_CLASSES]
    out_ref[...] = y * nd[:, None] + b2_ref[...]


def _deg_spec():
    return pl.BlockSpec((NC, ROW_BLK), lambda i: (0, i))


def _slab_spec(p, w=128):
    return pl.BlockSpec((p, ROW_BLK, w), lambda i: (0, i, 0))


def _full_spec(shape):
    nd = len(shape)
    return pl.BlockSpec(shape, lambda i: (0,) * nd)


def kernel(features, edge_index, W0, b0, W1, b1, W2, b2):
    ei = edge_index.astype(jnp.int32)
    src = ei[0]
    dst = ei[1]
    src_t = src.reshape(NS, NB, 80)
    dst_t = dst.reshape(NS, NB, 80)
    sd_a = jnp.stack([src.reshape(NS, 125, 80), dst.reshape(NS, 125, 80)],
                     axis=2)
    sd_s = jnp.stack([src.reshape(NC, NS, 125, 40),
                      dst.reshape(NC, NS, 125, 40)], axis=3)
    z1 = jnp.zeros((STRIPE,), jnp.float32)
    z128 = jnp.zeros((STRIPE, 128), jnp.float32)
    b0r = b0.reshape(1, N_HIDDEN)
    b1r = b1.reshape(1, N_HIDDEN)
    b2r = b2.reshape(1, N_CLASSES)
    featp = jnp.pad(features, ((0, N_PAD - N_NODES), (0, 0)))

    deg = _deg_kernel(src_t, dst_t, z1)

    h0s = pl.pallas_call(
        _tc_a_body,
        grid=(N_BLKS,),
        in_specs=[_deg_spec(), pl.BlockSpec((ROW_BLK, IN_FEATS), lambda i: (i, 0))],
        out_specs=_slab_spec(2),
        out_shape=jax.ShapeDtypeStruct((2, N_PAD, 128), jnp.float32),
    )(deg, featp)

    agg0 = _agg_256(h0s, sd_a, z128)

    h1s = pl.pallas_call(
        _tc_b_body,
        grid=(N_BLKS,),
        in_specs=[_deg_spec(), _slab_spec(2),
                  _full_spec((IN_FEATS, N_HIDDEN)), _full_spec((1, N_HIDDEN))],
        out_specs=_slab_spec(4),
        out_shape=jax.ShapeDtypeStruct((4, N_PAD, 128), jnp.float32),
    )(deg, agg0, W0, b0r)

    agg1 = _agg_512(h1s, sd_a, z128)

    z2 = pl.pallas_call(
        _tc_c_body,
        grid=(N_BLKS,),
        in_specs=[_deg_spec(), _slab_spec(4),
                  _full_spec((N_HIDDEN, N_HIDDEN)), _full_spec((1, N_HIDDEN)),
                  _full_spec((N_HIDDEN, N_CLASSES))],
        out_specs=_slab_spec(1),
        out_shape=jax.ShapeDtypeStruct((1, N_PAD, 128), jnp.float32),
    )(deg, agg1, W1, b1r, W2)

    agg2 = _agg_64(z2, sd_s, z128)

    out = pl.pallas_call(
        _tc_d_body,
        grid=(N_BLKS,),
        in_specs=[_deg_spec(), _slab_spec(2), _full_spec((1, N_CLASSES))],
        out_specs=pl.BlockSpec((ROW_BLK, N_CLASSES), lambda i: (i, 0)),
        out_shape=jax.ShapeDtypeStruct((N_PAD, N_CLASSES), jnp.float32),
    )(deg, agg2, b2r)

    return out[:N_NODES]


# speedup vs baseline: 1.0402x; 1.0402x over previous
import functools

import jax
import jax.numpy as jnp
from jax import lax
from jax.experimental import pallas as pl
from jax.experimental.pallas import tpu as pltpu
from jax.experimental.pallas import tpu_sc as plsc

N_NODES = 10000
N_EDGES = 160000
IN_FEATS = 256
N_HIDDEN = 512
N_CLASSES = 64

NC = 2
NS = 16
NB = 125
N_PAD = 10240
STRIPE = N_PAD // NS

ROW_BLK = 1024
N_BLKS = N_PAD // ROW_BLK

_SC_MESH = dict(core_axis_name="c", subcore_axis_name="s")


def _deg_kernel_body(src_t, dst_t, zeros_hbm, out_ref, idx_v, ones_v, hist_sp):
    c = lax.axis_index("c")
    s = lax.axis_index("s")

    @pl.when(c == 0)
    def _():
        pltpu.sync_copy(src_t.at[s], idx_v)

    @pl.when(c == 1)
    def _():
        pltpu.sync_copy(dst_t.at[s], idx_v)

    def fill_ones(i, _):
        ones_v[pl.ds(16 * i, 16)] = jnp.full((16,), 1.0, dtype=jnp.float32)
        return 0

    lax.fori_loop(0, 80 // 16, fill_ones, 0)

    pltpu.sync_copy(zeros_hbm, hist_sp.at[pl.ds(STRIPE * s, STRIPE)])
    plsc.subcore_barrier()

    def add_block(j, _):
        pltpu.sync_copy(ones_v, hist_sp.at[idx_v.at[j]], add=True)
        return 0

    lax.fori_loop(0, NB, add_block, 0)
    plsc.subcore_barrier()
    pltpu.sync_copy(hist_sp.at[pl.ds(STRIPE * s, STRIPE)],
                    out_ref.at[c].at[pl.ds(STRIPE * s, STRIPE)])


_deg_kernel = functools.partial(
    pl.kernel,
    out_type=jax.ShapeDtypeStruct((NC, N_PAD), jnp.float32),
    mesh=plsc.VectorSubcoreMesh(**_SC_MESH),
    scratch_types=[
        pltpu.VMEM((NB, 80), jnp.int32),
        pltpu.VMEM((80,), jnp.float32),
        pltpu.VMEM_SHARED((N_PAD,), jnp.float32),
    ],
)(_deg_kernel_body)


def _make_agg(P, split):
    W = 128
    R = 1 if split else P // NC
    BB = 40
    NBK = 125 if split else 250
    K = 9
    G = 5
    S = 2
    I = 7

    def body(h_ref, sd_ref, zeros_hbm, out_ref, idx_v, gbuf, acc,
             isem, gsem, ssem):
        c = lax.axis_index("c")
        s = lax.axis_index("s")
        sd_me = sd_ref.at[c].at[s] if split else sd_ref.at[s]

        def idx_start(j, buf):
            pltpu.make_async_copy(sd_me.at[j], idx_v.at[buf],
                                  isem.at[buf]).start()

        def idx_wait(j, buf):
            pltpu.make_async_copy(sd_me.at[j], idx_v.at[buf],
                                  isem.at[buf]).wait()

        for r in range(R):
            slab = 0 if split else NC * r + c
            pltpu.sync_copy(zeros_hbm, acc.at[pl.ds(STRIPE * s, STRIPE)])
            plsc.subcore_barrier()

            h_slab = h_ref.at[slab]

            def gather_start(j, buf):
                pltpu.make_async_copy(
                    h_slab.at[idx_v.at[buf].at[0]],
                    gbuf.at[buf], gsem.at[buf]).start()

            def gather_wait(j, buf):
                pltpu.make_async_copy(
                    h_slab.at[idx_v.at[buf].at[0]],
                    gbuf.at[buf], gsem.at[buf]).wait()

            def scatter_start(j, buf):
                pltpu.async_copy(gbuf.at[buf], acc.at[idx_v.at[buf].at[1]],
                                 ssem.at[buf], add=True)

            def scatter_wait(j, buf):
                pltpu.make_async_copy(gbuf.at[buf], acc.at[idx_v.at[buf].at[1]],
                                      ssem.at[buf]).wait()

            for t in range(I):
                idx_start(t, t)
            for t in range(G):
                idx_wait(t, t)
                gather_start(t, t)

            def step(j, _):
                buf = lax.rem(j, K)
                gather_wait(j, buf)
                scatter_start(j, buf)

                @pl.when(j >= S)
                def _():
                    scatter_wait(j - S, lax.rem(j + K - S, K))

                @pl.when(j + G < NBK)
                def _():
                    bg = lax.rem(j + G, K)
                    idx_wait(j + G, bg)
                    gather_start(j + G, bg)

                @pl.when(j + I < NBK)
                def _():
                    idx_start(j + I, lax.rem(j + I, K))

                return 0

            lax.fori_loop(0, NBK, step, 0)
            for t in range(S):
                scatter_wait(NBK - S + t, (NBK - S + t) % K)
            plsc.subcore_barrier()
            dst_slab = out_ref.at[c] if split else out_ref.at[slab]
            pltpu.sync_copy(acc.at[pl.ds(STRIPE * s, STRIPE)],
                            dst_slab.at[pl.ds(STRIPE * s, STRIPE)])
            if r + 1 < R:
                plsc.subcore_barrier()

    out_slabs = NC if split else P
    idx_shape = (NC, NS) if split else (NS,)
    return functools.partial(
        pl.kernel,
        out_type=jax.ShapeDtypeStruct((out_slabs, N_PAD, W), jnp.float32),
        mesh=plsc.VectorSubcoreMesh(**_SC_MESH),
        scratch_types=[
            pltpu.VMEM((K, 2, BB), jnp.int32),
            pltpu.VMEM((K, BB, W), jnp.float32),
            pltpu.VMEM_SHARED((N_PAD, W), jnp.float32),
            pltpu.SemaphoreType.DMA((K,)),
            pltpu.SemaphoreType.DMA((K,)),
            pltpu.SemaphoreType.DMA((K,)),
        ],
    )(body)


_agg_256 = _make_agg(2, split=False)
_agg_512 = _make_agg(4, split=False)
_agg_64 = _make_agg(1, split=True)


def _norms(deg_ref):
    dsrc = deg_ref[0, :]
    ddst = deg_ref[1, :]
    ns = jnp.where(dsrc > 0, lax.rsqrt(jnp.maximum(dsrc, 1.0)), 0.0)
    nd = jnp.where(ddst > 0, lax.rsqrt(jnp.maximum(ddst, 1.0)), 0.0)
    return ns, nd


def _tc_a_body(deg_ref, feat_ref, out_ref):
    ns, _ = _norms(deg_ref)
    x = feat_ref[...] * ns[:, None]
    out_ref[0] = x[:, 0:128]
    out_ref[1] = x[:, 128:256]


def _tc_b_body(deg_ref, agg_ref, w0_ref, b0_ref, out_ref):
    ns, nd = _norms(deg_ref)
    acc = jnp.dot(agg_ref[0], w0_ref[0:128, :], preferred_element_type=jnp.float32)
    acc += jnp.dot(agg_ref[1], w0_ref[128:256, :], preferred_element_type=jnp.float32)
    h1 = jnp.maximum(acc * nd[:, None] + b0_ref[...], 0.0) * ns[:, None]
    for q in range(4):
        out_ref[q] = h1[:, 128 * q:128 * (q + 1)]


def _tc_c_body(deg_ref, agg_ref, w1_ref, b1_ref, w2_ref, out_ref):
    ns, nd = _norms(deg_ref)
    acc = jnp.dot(agg_ref[0], w1_ref[0:128, :], preferred_element_type=jnp.float32)
    for p in range(1, 4):
        acc += jnp.dot(agg_ref[p], w1_ref[128 * p:128 * (p + 1), :],
                       preferred_element_type=jnp.float32)
    t = jnp.maximum(acc * nd[:, None] + b1_ref[...], 0.0) * ns[:, None]
    z = jnp.dot(t, w2_ref[...], preferred_element_type=jnp.float32)
    out_ref[0] = jnp.concatenate([z, jnp.zeros_like(z)], axis=1)


def _tc_d_body(deg_ref, agg_ref, b2_ref, out_ref):
    _, nd = _norms(deg_ref)
    y = agg_ref[0, :, 0:N_CLASSES] + agg_ref[1, :, 0:N_CLASSES]
    out_ref[...] = y * nd[:, None] + b2_ref[...]


def _deg_spec():
    return pl.BlockSpec((NC, ROW_BLK), lambda i: (0, i))


def _slab_spec(p, w=128):
    return pl.BlockSpec((p, ROW_BLK, w), lambda i: (0, i, 0))


def _full_spec(shape):
    nd = len(shape)
    return pl.BlockSpec(shape, lambda i: (0,) * nd)


def kernel(features, edge_index, W0, b0, W1, b1, W2, b2):
    ei = edge_index.astype(jnp.int32)
    src = ei[0]
    dst = ei[1]
    src_t = src.reshape(NS, NB, 80)
    dst_t = dst.reshape(NS, NB, 80)
    sd_a = jnp.stack([src.reshape(NS, 250, 40), dst.reshape(NS, 250, 40)],
                     axis=2)
    sd_s = jnp.stack([src.reshape(NC, NS, 125, 40),
                      dst.reshape(NC, NS, 125, 40)], axis=3)
    z1 = jnp.zeros((STRIPE,), jnp.float32)
    z128 = jnp.zeros((STRIPE, 128), jnp.float32)
    b0r = b0.reshape(1, N_HIDDEN)
    b1r = b1.reshape(1, N_HIDDEN)
    b2r = b2.reshape(1, N_CLASSES)
    featp = jnp.pad(features, ((0, N_PAD - N_NODES), (0, 0)))

    deg = _deg_kernel(src_t, dst_t, z1)

    h0s = pl.pallas_call(
        _tc_a_body,
        grid=(N_BLKS,),
        in_specs=[_deg_spec(), pl.BlockSpec((ROW_BLK, IN_FEATS), lambda i: (i, 0))],
        out_specs=_slab_spec(2),
        out_shape=jax.ShapeDtypeStruct((2, N_PAD, 128), jnp.float32),
    )(deg, featp)

    agg0 = _agg_256(h0s, sd_a, z128)

    h1s = pl.pallas_call(
        _tc_b_body,
        grid=(N_BLKS,),
        in_specs=[_deg_spec(), _slab_spec(2),
                  _full_spec((IN_FEATS, N_HIDDEN)), _full_spec((1, N_HIDDEN))],
        out_specs=_slab_spec(4),
        out_shape=jax.ShapeDtypeStruct((4, N_PAD, 128), jnp.float32),
    )(deg, agg0, W0, b0r)

    agg1 = _agg_512(h1s, sd_a, z128)

    z2 = pl.pallas_call(
        _tc_c_body,
        grid=(N_BLKS,),
        in_specs=[_deg_spec(), _slab_spec(4),
                  _full_spec((N_HIDDEN, N_HIDDEN)), _full_spec((1, N_HIDDEN)),
                  _full_spec((N_HIDDEN, N_CLASSES))],
        out_specs=_slab_spec(1),
        out_shape=jax.ShapeDtypeStruct((1, N_PAD, 128), jnp.float32),
    )(deg, agg1, W1, b1r, W2)

    agg2 = _agg_64(z2, sd_s, z128)

    out = pl.pallas_call(
        _tc_d_body,
        grid=(N_BLKS,),
        in_specs=[_deg_spec(), _slab_spec(2), _full_spec((1, N_CLASSES))],
        out_specs=pl.BlockSpec((ROW_BLK, N_CLASSES), lambda i: (i, 0)),
        out_shape=jax.ShapeDtypeStruct((N_PAD, N_CLASSES), jnp.float32),
    )(deg, agg2, b2r)

    return out[:N_NODES]


# speedup vs baseline: 1.0465x; 1.0061x over previous
import functools

import jax
import jax.numpy as jnp
from jax import lax
from jax.experimental import pallas as pl
from jax.experimental.pallas import tpu as pltpu
from jax.experimental.pallas import tpu_sc as plsc

N_NODES = 10000
N_EDGES = 160000
IN_FEATS = 256
N_HIDDEN = 512
N_CLASSES = 64

NC = 2
NS = 16
NB = 125
N_PAD = 10240
STRIPE = N_PAD // NS

ROW_BLK = 1024
N_BLKS = N_PAD // ROW_BLK

_SC_MESH = dict(core_axis_name="c", subcore_axis_name="s")


def _deg_kernel_body(src_t, dst_t, zeros_hbm, out_ref, idx_v, ones_v, hist_sp):
    c = lax.axis_index("c")
    s = lax.axis_index("s")

    @pl.when(c == 0)
    def _():
        pltpu.sync_copy(src_t.at[s], idx_v)

    @pl.when(c == 1)
    def _():
        pltpu.sync_copy(dst_t.at[s], idx_v)

    def fill_ones(i, _):
        ones_v[pl.ds(16 * i, 16)] = jnp.full((16,), 1.0, dtype=jnp.float32)
        return 0

    lax.fori_loop(0, 80 // 16, fill_ones, 0)

    pltpu.sync_copy(zeros_hbm, hist_sp.at[pl.ds(STRIPE * s, STRIPE)])
    plsc.subcore_barrier()

    def add_block(j, _):
        pltpu.sync_copy(ones_v, hist_sp.at[idx_v.at[j]], add=True)
        return 0

    lax.fori_loop(0, NB, add_block, 0)
    plsc.subcore_barrier()
    pltpu.sync_copy(hist_sp.at[pl.ds(STRIPE * s, STRIPE)],
                    out_ref.at[c].at[pl.ds(STRIPE * s, STRIPE)])


_deg_kernel = functools.partial(
    pl.kernel,
    out_type=jax.ShapeDtypeStruct((NC, N_PAD), jnp.float32),
    mesh=plsc.VectorSubcoreMesh(**_SC_MESH),
    scratch_types=[
        pltpu.VMEM((NB, 80), jnp.int32),
        pltpu.VMEM((80,), jnp.float32),
        pltpu.VMEM_SHARED((N_PAD,), jnp.float32),
    ],
)(_deg_kernel_body)


def _make_agg(P, split, dtype=jnp.float32):
    W = 128
    R = 1 if split else P // NC
    BB = 40
    NBK = 125 if split else 250
    K = 9
    G = 5
    S = 2
    I = 7

    def body(h_ref, sd_ref, zeros_hbm, out_ref, idx_v, gbuf, acc,
             isem, gsem, ssem):
        c = lax.axis_index("c")
        s = lax.axis_index("s")
        sd_me = sd_ref.at[c].at[s] if split else sd_ref.at[s]

        def idx_start(j, buf):
            pltpu.make_async_copy(sd_me.at[j], idx_v.at[buf],
                                  isem.at[buf]).start()

        def idx_wait(j, buf):
            pltpu.make_async_copy(sd_me.at[j], idx_v.at[buf],
                                  isem.at[buf]).wait()

        for r in range(R):
            slab = 0 if split else NC * r + c
            h_slab = h_ref.at[slab]

            def gather_start(j, buf):
                pltpu.make_async_copy(
                    h_slab.at[idx_v.at[buf].at[0]],
                    gbuf.at[buf], gsem.at[buf]).start()

            def gather_wait(j, buf):
                pltpu.make_async_copy(
                    h_slab.at[idx_v.at[buf].at[0]],
                    gbuf.at[buf], gsem.at[buf]).wait()

            def scatter_start(j, buf):
                pltpu.async_copy(gbuf.at[buf], acc.at[idx_v.at[buf].at[1]],
                                 ssem.at[buf], add=True)

            def scatter_wait(j, buf):
                pltpu.make_async_copy(gbuf.at[buf], acc.at[idx_v.at[buf].at[1]],
                                      ssem.at[buf]).wait()

            for t in range(I):
                idx_start(t, t)
            pltpu.sync_copy(zeros_hbm, acc.at[pl.ds(STRIPE * s, STRIPE)])
            for t in range(G):
                idx_wait(t, t)
                gather_start(t, t)
            plsc.subcore_barrier()

            def step(j, _):
                buf = lax.rem(j, K)
                gather_wait(j, buf)
                scatter_start(j, buf)

                @pl.when(j >= S)
                def _():
                    scatter_wait(j - S, lax.rem(j + K - S, K))

                @pl.when(j + G < NBK)
                def _():
                    bg = lax.rem(j + G, K)
                    idx_wait(j + G, bg)
                    gather_start(j + G, bg)

                @pl.when(j + I < NBK)
                def _():
                    idx_start(j + I, lax.rem(j + I, K))

                return 0

            lax.fori_loop(0, NBK, step, 0)
            for t in range(S):
                scatter_wait(NBK - S + t, (NBK - S + t) % K)
            plsc.subcore_barrier()
            dst_slab = out_ref.at[c] if split else out_ref.at[slab]
            pltpu.sync_copy(acc.at[pl.ds(STRIPE * s, STRIPE)],
                            dst_slab.at[pl.ds(STRIPE * s, STRIPE)])
            if r + 1 < R:
                plsc.subcore_barrier()

    out_slabs = NC if split else P
    idx_shape = (NC, NS) if split else (NS,)
    return functools.partial(
        pl.kernel,
        out_type=jax.ShapeDtypeStruct((out_slabs, N_PAD, W), dtype),
        mesh=plsc.VectorSubcoreMesh(**_SC_MESH),
        scratch_types=[
            pltpu.VMEM((K, 2, BB), jnp.int32),
            pltpu.VMEM((K, BB, W), dtype),
            pltpu.VMEM_SHARED((N_PAD, W), dtype),
            pltpu.SemaphoreType.DMA((K,)),
            pltpu.SemaphoreType.DMA((K,)),
            pltpu.SemaphoreType.DMA((K,)),
        ],
    )(body)


_agg_256 = _make_agg(2, split=False)
_agg_512 = _make_agg(4, split=False)
_agg_64 = _make_agg(1, split=True)


def _norms(deg_ref):
    dsrc = deg_ref[0, :]
    ddst = deg_ref[1, :]
    ns = jnp.where(dsrc > 0, lax.rsqrt(jnp.maximum(dsrc, 1.0)), 0.0)
    nd = jnp.where(ddst > 0, lax.rsqrt(jnp.maximum(ddst, 1.0)), 0.0)
    return ns, nd


def _tc_a_body(deg_ref, feat_ref, out_ref):
    ns, _ = _norms(deg_ref)
    x = feat_ref[...] * ns[:, None]
    out_ref[0] = x[:, 0:128]
    out_ref[1] = x[:, 128:256]


def _tc_b_body(deg_ref, agg_ref, w0_ref, b0_ref, out_ref):
    ns, nd = _norms(deg_ref)
    acc = jnp.dot(agg_ref[0], w0_ref[0:128, :], preferred_element_type=jnp.float32)
    acc += jnp.dot(agg_ref[1], w0_ref[128:256, :], preferred_element_type=jnp.float32)
    h1 = jnp.maximum(acc * nd[:, None] + b0_ref[...], 0.0) * ns[:, None]
    for q in range(4):
        out_ref[q] = h1[:, 128 * q:128 * (q + 1)]


def _tc_c_body(deg_ref, agg_ref, w1_ref, b1_ref, w2_ref, out_ref):
    ns, nd = _norms(deg_ref)
    acc = jnp.dot(agg_ref[0], w1_ref[0:128, :], preferred_element_type=jnp.float32)
    for p in range(1, 4):
        acc += jnp.dot(agg_ref[p], w1_ref[128 * p:128 * (p + 1), :],
                       preferred_element_type=jnp.float32)
    t = jnp.maximum(acc * nd[:, None] + b1_ref[...], 0.0) * ns[:, None]
    z = jnp.dot(t, w2_ref[...], preferred_element_type=jnp.float32)
    out_ref[0] = jnp.concatenate([z, jnp.zeros_like(z)], axis=1)


def _tc_d_body(deg_ref, agg_ref, b2_ref, out_ref):
    _, nd = _norms(deg_ref)
    y = agg_ref[0, :, 0:N_CLASSES] + agg_ref[1, :, 0:N_CLASSES]
    out_ref[...] = y * nd[:, None] + b2_ref[...]


def _deg_spec():
    return pl.BlockSpec((NC, ROW_BLK), lambda i: (0, i))


def _slab_spec(p, w=128):
    return pl.BlockSpec((p, ROW_BLK, w), lambda i: (0, i, 0))


def _full_spec(shape):
    nd = len(shape)
    return pl.BlockSpec(shape, lambda i: (0,) * nd)


def kernel(features, edge_index, W0, b0, W1, b1, W2, b2):
    ei = edge_index.astype(jnp.int32)
    src = ei[0]
    dst = ei[1]
    src_t = src.reshape(NS, NB, 80)
    dst_t = dst.reshape(NS, NB, 80)
    sd_a = jnp.stack([src.reshape(NS, 250, 40), dst.reshape(NS, 250, 40)],
                     axis=2)
    sd_s = jnp.stack([src.reshape(NC, NS, 125, 40),
                      dst.reshape(NC, NS, 125, 40)], axis=3)
    z1 = jnp.zeros((STRIPE,), jnp.float32)
    z128 = jnp.zeros((STRIPE, 128), jnp.float32)
    b0r = b0.reshape(1, N_HIDDEN)
    b1r = b1.reshape(1, N_HIDDEN)
    b2r = b2.reshape(1, N_CLASSES)
    featp = jnp.pad(features, ((0, N_PAD - N_NODES), (0, 0)))

    deg = _deg_kernel(src_t, dst_t, z1)

    h0s = pl.pallas_call(
        _tc_a_body,
        grid=(N_BLKS,),
        in_specs=[_deg_spec(), pl.BlockSpec((ROW_BLK, IN_FEATS), lambda i: (i, 0))],
        out_specs=_slab_spec(2),
        out_shape=jax.ShapeDtypeStruct((2, N_PAD, 128), jnp.float32),
    )(deg, featp)

    agg0 = _agg_256(h0s, sd_a, z128)

    h1s = pl.pallas_call(
        _tc_b_body,
        grid=(N_BLKS,),
        in_specs=[_deg_spec(), _slab_spec(2),
                  _full_spec((IN_FEATS, N_HIDDEN)), _full_spec((1, N_HIDDEN))],
        out_specs=_slab_spec(4),
        out_shape=jax.ShapeDtypeStruct((4, N_PAD, 128), jnp.float32),
    )(deg, agg0, W0, b0r)

    agg1 = _agg_512(h1s, sd_a, z128)

    z2 = pl.pallas_call(
        _tc_c_body,
        grid=(N_BLKS,),
        in_specs=[_deg_spec(), _slab_spec(4),
                  _full_spec((N_HIDDEN, N_HIDDEN)), _full_spec((1, N_HIDDEN)),
                  _full_spec((N_HIDDEN, N_CLASSES))],
        out_specs=_slab_spec(1),
        out_shape=jax.ShapeDtypeStruct((1, N_PAD, 128), jnp.float32),
    )(deg, agg1, W1, b1r, W2)

    agg2 = _agg_64(z2, sd_s, z128)

    out = pl.pallas_call(
        _tc_d_body,
        grid=(N_BLKS,),
        in_specs=[_deg_spec(), _slab_spec(2), _full_spec((1, N_CLASSES))],
        out_specs=pl.BlockSpec((ROW_BLK, N_CLASSES), lambda i: (i, 0)),
        out_shape=jax.ShapeDtypeStruct((N_PAD, N_CLASSES), jnp.float32),
    )(deg, agg2, b2r)

    return out[:N_NODES]
